# Initial kernel scaffold; baseline (speedup 1.0000x reference)
import jax
import jax.numpy as jnp
from jax.experimental import pallas as pl


def kernel(adj_indices, adj_values, user_table, item_table):
    raise NotImplementedError("write your pallas kernel here")


# speedup vs baseline: 3.3797x; 3.3797x over previous
import jax
import jax.numpy as jnp
from jax import lax
from jax.experimental import pallas as pl
from jax.experimental.pallas import tpu as pltpu
from jax.experimental.pallas import tpu_sc as plsc

N_USERS = 25000
N_ITEMS = 25000
N_NODES = N_USERS + N_ITEMS
LATENT = 64
N_EDGES = 800000

NC = 2
NS = 16
HALF = LATENT // NC

CHUNK = 128
PER_TILE = -(-N_EDGES // (NS * CHUNK)) * CHUNK
E_PAD = PER_TILE * NS
N_CHUNKS = PER_TILE // CHUNK

N_PAD = 50048
ROWS_PER_TILE = N_PAD // NS
ZROWS = 782


def _layer_body(tab, src_hbm, dst_hbm, val_hbm, out,
                acc, zbuf, sidx, didx, vbuf, rows, gsem, ssem):
    c = lax.axis_index("c")
    s = lax.axis_index("s")

    def _zero(j, carry):
        zbuf[j, pl.ds(0, 16)] = jnp.zeros((16,), jnp.float32)
        zbuf[j, pl.ds(16, 16)] = jnp.zeros((16,), jnp.float32)
        return carry
    lax.fori_loop(0, ZROWS, _zero, 0, unroll=4)
    for k in range(ROWS_PER_TILE // ZROWS):
        pltpu.sync_copy(zbuf, acc.at[pl.ds(s * ROWS_PER_TILE + k * ZROWS, ZROWS)])
    plsc.subcore_barrier()

    sbase = c * E_PAD + s * PER_TILE
    base = s * PER_TILE

    def _chunk(i, carry):
        pltpu.sync_copy(src_hbm.at[pl.ds(sbase + i * CHUNK, CHUNK)], sidx)
        pltpu.sync_copy(dst_hbm.at[pl.ds(base + i * CHUNK, CHUNK)], didx)
        pltpu.sync_copy(val_hbm.at[pl.ds(base + i * CHUNK, CHUNK)], vbuf)

        pltpu.async_copy(tab.at[sidx], rows, gsem).wait()

        def _scale(g, carry2):
            vv = vbuf[pl.ds(g * 16, 16)]
            for e in range(16):
                b = vv.at[jnp.full((16,), e, jnp.int32)].get(
                    mode="promise_in_bounds")
                r = g * 16 + e
                rows[r, pl.ds(0, 16)] = rows[r, pl.ds(0, 16)] * b
                rows[r, pl.ds(16, 16)] = rows[r, pl.ds(16, 16)] * b
            return carry2
        lax.fori_loop(0, CHUNK // 16, _scale, 0)

        pltpu.async_copy(rows, acc.at[didx], ssem, add=True).wait()
        return carry

    lax.fori_loop(0, N_CHUNKS, _chunk, 0)
    plsc.subcore_barrier()

    pltpu.sync_copy(acc.at[pl.ds(s * ROWS_PER_TILE, ROWS_PER_TILE)],
                    out.at[pl.ds(c * N_PAD + s * ROWS_PER_TILE, ROWS_PER_TILE)])


_layer = pl.kernel(
    _layer_body,
    out_type=jax.ShapeDtypeStruct((NC * N_PAD, HALF), jnp.float32),
    mesh=plsc.VectorSubcoreMesh(core_axis_name="c", subcore_axis_name="s",
                                num_cores=NC, num_subcores=NS),
    compiler_params=pltpu.CompilerParams(use_tc_tiling_on_sc=False),
    scratch_types=[
        pltpu.VMEM_SHARED((N_PAD, HALF), jnp.float32),
        pltpu.VMEM((ZROWS, HALF), jnp.float32),
        pltpu.VMEM((CHUNK,), jnp.int32),
        pltpu.VMEM((CHUNK,), jnp.int32),
        pltpu.VMEM((CHUNK,), jnp.float32),
        pltpu.VMEM((CHUNK, HALF), jnp.float32),
        pltpu.SemaphoreType.DMA,
        pltpu.SemaphoreType.DMA,
    ],
)


def _mean_body(t0, l1, l2, l3, out):
    out[...] = (t0[...] + l1[...] + l2[...] + l3[...]) * 0.25


_MBLK = 2176
_HOFF = N_PAD // _MBLK


def _mean(t0, l1, l2, l3, half):
    spec = pl.BlockSpec((_MBLK, HALF), lambda i, h=half: (i + h * _HOFF, 0))
    return pl.pallas_call(
        _mean_body,
        grid=(_HOFF,),
        in_specs=[spec] * 4,
        out_specs=pl.BlockSpec((_MBLK, HALF), lambda i: (i, 0)),
        out_shape=jax.ShapeDtypeStruct((N_PAD, HALF), jnp.float32),
    )(t0, l1, l2, l3)


def kernel(adj_indices, adj_values, user_table, item_table):
    dst = adj_indices[0].astype(jnp.int32)
    src = adj_indices[1].astype(jnp.int32)
    pad = E_PAD - N_EDGES
    src = jnp.pad(src, (0, pad))
    src2 = jnp.concatenate([src, src + N_PAD])
    dst = jnp.pad(dst, (0, pad))
    val = jnp.pad(adj_values, (0, pad))

    zpad = jnp.zeros((N_PAD - N_NODES, HALF), jnp.float32)
    t0 = jnp.concatenate([user_table[:, :HALF], item_table[:, :HALF], zpad,
                          user_table[:, HALF:], item_table[:, HALF:], zpad],
                         axis=0)

    l1 = _layer(t0, src2, dst, val)
    l2 = _layer(l1, src2, dst, val)
    l3 = _layer(l2, src2, dst, val)

    m0 = _mean(t0, l1, l2, l3, 0)
    m1 = _mean(t0, l1, l2, l3, 1)
    emb = jnp.concatenate([m0[:N_NODES], m1[:N_NODES]], axis=1)
    return emb[:N_USERS], emb[N_USERS:N_NODES]


# speedup vs baseline: 7.8122x; 2.3115x over previous
import jax
import jax.numpy as jnp
from jax import lax
from jax.experimental import pallas as pl
from jax.experimental.pallas import tpu as pltpu
from jax.experimental.pallas import tpu_sc as plsc

N_USERS = 25000
N_ITEMS = 25000
N_NODES = N_USERS + N_ITEMS
LATENT = 64
N_EDGES = 800000

NC = 2
NS = 16
HALF = LATENT // NC

CHUNK = 128
DEPTH = 4
N_CHUNKS = -(-N_EDGES // (NS * CHUNK * DEPTH)) * DEPTH
PER_TILE = N_CHUNKS * CHUNK
E_PAD = PER_TILE * NS
CPT = E_PAD // CHUNK

N_PAD = 50048
ROWS_PER_TILE = N_PAD // NS

_GRP = CHUNK // 16
_ZFULL = ROWS_PER_TILE // CHUNK
_ZTAIL = ROWS_PER_TILE - _ZFULL * CHUNK


def _layer_body(tab, packed, out,
                acc,
                pbuf0, pbuf1, pbuf2, pbuf3,
                rows0, rows1, rows2, rows3,
                didx0, didx1, didx2, didx3,
                isem0, isem1, isem2, isem3,
                gsem0, gsem1, gsem2, gsem3,
                ssem0, ssem1, ssem2, ssem3):
    c = lax.axis_index("c")
    s = lax.axis_index("s")
    pbuf = (pbuf0, pbuf1, pbuf2, pbuf3)
    rows = (rows0, rows1, rows2, rows3)
    didx = (didx0, didx1, didx2, didx3)
    isem = (isem0, isem1, isem2, isem3)
    gsem = (gsem0, gsem1, gsem2, gsem3)
    ssem = (ssem0, ssem1, ssem2, ssem3)

    def _zero(j, carry):
        for b in range(DEPTH):
            rows[b][j, pl.ds(0, 16)] = jnp.zeros((16,), jnp.float32)
            rows[b][j, pl.ds(16, 16)] = jnp.zeros((16,), jnp.float32)
        return carry
    lax.fori_loop(0, CHUNK, _zero, 0, unroll=2)
    slab = s * ROWS_PER_TILE
    for k in range(_ZFULL):
        pltpu.async_copy(rows[k % DEPTH], acc.at[pl.ds(slab + k * CHUNK, CHUNK)],
                         gsem[k % DEPTH])
    for k in range(_ZFULL):
        pltpu.make_async_copy(rows[k % DEPTH],
                              acc.at[pl.ds(slab + k * CHUNK, CHUNK)],
                              gsem[k % DEPTH]).wait()
    pltpu.sync_copy(rows[0].at[pl.ds(0, _ZTAIL)],
                    acc.at[pl.ds(slab + _ZFULL * CHUNK, _ZTAIL)])
    plsc.subcore_barrier()

    cbase = c * CPT + s * N_CHUNKS

    def load_meta(i, b):
        pltpu.async_copy(packed.at[cbase + i], pbuf[b], isem[b])

    def wait_meta(i, b):
        pltpu.make_async_copy(packed.at[cbase + i], pbuf[b], isem[b]).wait()

    def start_gather(b):
        pltpu.async_copy(tab.at[pbuf[b].at[0]], rows[b], gsem[b])

    def wait_gather(b):
        pltpu.make_async_copy(tab.at[pbuf[b].at[0]], rows[b], gsem[b]).wait()

    def wait_scatter(b):
        pltpu.make_async_copy(rows[b], acc.at[didx[b]], ssem[b]).wait()

    def scale(b):
        pb, rb, db = pbuf[b], rows[b], didx[b]

        def _grp(g, carry):
            sl = pl.ds(g * 16, 16)
            db[sl] = pb[1, sl]
            vv = plsc.bitcast(pb[2, sl], jnp.float32)
            for e in range(16):
                bc = vv.at[jnp.full((16,), e, jnp.int32)].get(
                    mode="promise_in_bounds")
                r = g * 16 + e
                rb[r, pl.ds(0, 16)] = rb[r, pl.ds(0, 16)] * bc
                rb[r, pl.ds(16, 16)] = rb[r, pl.ds(16, 16)] * bc
            return carry
        lax.fori_loop(0, _GRP, _grp, 0)

    def step(i, b, *, do_next_gather=True, do_meta=True, wait_sc=True,
             sync_scatter=False):
        b1 = (b + 1) % DEPTH
        wait_gather(b)
        if wait_sc:
            wait_scatter(b1)
        if do_next_gather:
            wait_meta(i + 1, b1)
            start_gather(b1)
        scale(b)
        if sync_scatter:
            pltpu.sync_copy(rows[b], acc.at[didx[b]], add=True)
        else:
            pltpu.async_copy(rows[b], acc.at[didx[b]], ssem[b], add=True)
        if do_meta:
            load_meta(i + DEPTH, b)

    for b in range(DEPTH):
        load_meta(b, b)
    wait_meta(0, 0)
    start_gather(0)
    for i in range(DEPTH):
        step(i, i, wait_sc=(i == DEPTH - 1))

    def _main(g, carry):
        i0 = DEPTH + g * DEPTH
        for b in range(DEPTH):
            step(i0 + b, b)
        return carry
    lax.fori_loop(0, (N_CHUNKS - 2 * DEPTH) // DEPTH, _main, 0)

    for k in range(DEPTH):
        i = N_CHUNKS - DEPTH + k
        step(i, i % DEPTH, do_next_gather=(k != DEPTH - 1), do_meta=False,
             wait_sc=(k != DEPTH - 1), sync_scatter=True)

    plsc.subcore_barrier()

    pltpu.sync_copy(acc.at[pl.ds(s * ROWS_PER_TILE, ROWS_PER_TILE)],
                    out.at[pl.ds(c * N_PAD + s * ROWS_PER_TILE, ROWS_PER_TILE)])


_layer = pl.kernel(
    _layer_body,
    out_type=jax.ShapeDtypeStruct((NC * N_PAD, HALF), jnp.float32),
    mesh=plsc.VectorSubcoreMesh(core_axis_name="c", subcore_axis_name="s",
                                num_cores=NC, num_subcores=NS),
    compiler_params=pltpu.CompilerParams(use_tc_tiling_on_sc=False,
                                         needs_layout_passes=False),
    scratch_types=(
        [pltpu.VMEM_SHARED((N_PAD, HALF), jnp.float32)]
        + [pltpu.VMEM((3, CHUNK), jnp.int32)] * DEPTH
        + [pltpu.VMEM((CHUNK, HALF), jnp.float32)] * DEPTH
        + [pltpu.VMEM((CHUNK,), jnp.int32)] * DEPTH
        + [pltpu.SemaphoreType.DMA] * (3 * DEPTH)
    ),
)


def _mean_body(t0, l1, l2, l3, out):
    out[...] = (t0[...] + l1[...] + l2[...] + l3[...]) * 0.25


_MBLK = 2176
_HOFF = N_PAD // _MBLK


def _mean(t0, l1, l2, l3, half):
    spec = pl.BlockSpec((_MBLK, HALF), lambda i, h=half: (i + h * _HOFF, 0))
    return pl.pallas_call(
        _mean_body,
        grid=(_HOFF,),
        in_specs=[spec] * 4,
        out_specs=pl.BlockSpec((_MBLK, HALF), lambda i: (i, 0)),
        out_shape=jax.ShapeDtypeStruct((N_PAD, HALF), jnp.float32),
    )(t0, l1, l2, l3)


def kernel(adj_indices, adj_values, user_table, item_table):
    dst = adj_indices[0].astype(jnp.int32)
    src = adj_indices[1].astype(jnp.int32)
    pad = E_PAD - N_EDGES
    src = jnp.pad(src, (0, pad))
    dst = jnp.pad(dst, (0, pad))
    val = jnp.pad(adj_values, (0, pad))
    valb = lax.bitcast_convert_type(val, jnp.int32)

    def mk(c):
        a = jnp.stack([src + c * N_PAD, dst, valb], axis=0)
        return a.reshape(3, CPT, CHUNK).transpose(1, 0, 2)
    packed = jnp.concatenate([mk(0), mk(1)], axis=0)

    zpad = jnp.zeros((N_PAD - N_NODES, HALF), jnp.float32)
    t0 = jnp.concatenate([user_table[:, :HALF], item_table[:, :HALF], zpad,
                          user_table[:, HALF:], item_table[:, HALF:], zpad],
                         axis=0)

    l1 = _layer(t0, packed)
    l2 = _layer(l1, packed)
    l3 = _layer(l2, packed)

    m0 = _mean(t0, l1, l2, l3, 0)
    m1 = _mean(t0, l1, l2, l3, 1)
    emb = jnp.concatenate([m0[:N_NODES], m1[:N_NODES]], axis=1)
    return emb[:N_USERS], emb[N_USERS:N_NODES]


# speedup vs baseline: 8.5656x; 1.0964x over previous
import jax
import jax.numpy as jnp
from jax import lax
from jax.experimental import pallas as pl
from jax.experimental.pallas import tpu as pltpu
from jax.experimental.pallas import tpu_sc as plsc

N_USERS = 25000
N_ITEMS = 25000
N_NODES = N_USERS + N_ITEMS
LATENT = 64
N_EDGES = 800000

NC = 2
NS = 16
HALF = LATENT // NC

CHUNK = 128
DEPTH = 6
GA = 2
N_CHUNKS = -(-N_EDGES // (NS * CHUNK * DEPTH)) * DEPTH
PER_TILE = N_CHUNKS * CHUNK
E_PAD = PER_TILE * NS
CPT = E_PAD // CHUNK

N_PAD = 50048
ROWS_PER_TILE = N_PAD // NS

_GRP = CHUNK // 16
_ZFULL = ROWS_PER_TILE // CHUNK
_ZTAIL = ROWS_PER_TILE - _ZFULL * CHUNK


def _layer_body(tab, packed, out, acc, *scratch):
    c = lax.axis_index("c")
    s = lax.axis_index("s")
    pbuf = scratch[0:DEPTH]
    rows = scratch[DEPTH:2 * DEPTH]
    didx = scratch[2 * DEPTH:3 * DEPTH]
    isem = scratch[3 * DEPTH:4 * DEPTH]
    gsem = scratch[4 * DEPTH:5 * DEPTH]
    ssem = scratch[5 * DEPTH:6 * DEPTH]

    def _zero(j, carry):
        for b in range(DEPTH):
            rows[b][j, pl.ds(0, 16)] = jnp.zeros((16,), jnp.float32)
            rows[b][j, pl.ds(16, 16)] = jnp.zeros((16,), jnp.float32)
        return carry
    lax.fori_loop(0, CHUNK, _zero, 0, unroll=2)
    slab = s * ROWS_PER_TILE
    for k in range(_ZFULL):
        pltpu.async_copy(rows[k % DEPTH], acc.at[pl.ds(slab + k * CHUNK, CHUNK)],
                         gsem[k % DEPTH])
    for k in range(_ZFULL):
        pltpu.make_async_copy(rows[k % DEPTH],
                              acc.at[pl.ds(slab + k * CHUNK, CHUNK)],
                              gsem[k % DEPTH]).wait()
    pltpu.sync_copy(rows[0].at[pl.ds(0, _ZTAIL)],
                    acc.at[pl.ds(slab + _ZFULL * CHUNK, _ZTAIL)])
    plsc.subcore_barrier()

    cbase = c * CPT + s * N_CHUNKS

    def load_meta(i, b):
        pltpu.async_copy(packed.at[cbase + i], pbuf[b], isem[b])

    def wait_meta(i, b):
        pltpu.make_async_copy(packed.at[cbase + i], pbuf[b], isem[b]).wait()

    def start_gather(b):
        pltpu.async_copy(tab.at[pbuf[b].at[0]], rows[b], gsem[b])

    def wait_gather(b):
        pltpu.make_async_copy(tab.at[pbuf[b].at[0]], rows[b], gsem[b]).wait()

    def wait_scatter(b):
        pltpu.make_async_copy(rows[b], acc.at[didx[b]], ssem[b]).wait()

    def scale(b):
        pb, rb, db = pbuf[b], rows[b], didx[b]

        def _grp(g, carry):
            sl = pl.ds(g * 16, 16)
            db[sl] = pb[1, sl]
            vv = plsc.bitcast(pb[2, sl], jnp.float32)
            for e in range(16):
                bc = vv.at[jnp.full((16,), e, jnp.int32)].get(
                    mode="promise_in_bounds")
                r = g * 16 + e
                rb[r, pl.ds(0, 16)] = rb[r, pl.ds(0, 16)] * bc
                rb[r, pl.ds(16, 16)] = rb[r, pl.ds(16, 16)] * bc
            return carry
        lax.fori_loop(0, _GRP, _grp, 0)

    def step(i, b, *, next_gather=True, do_meta=True, wait_sc=True,
             sync_scatter=False):
        b2 = (b + GA) % DEPTH
        wait_gather(b)
        if next_gather:
            if wait_sc:
                wait_scatter(b2)
            wait_meta(i + GA, b2)
            start_gather(b2)
        scale(b)
        if sync_scatter:
            pltpu.sync_copy(rows[b], acc.at[didx[b]], add=True)
        else:
            pltpu.async_copy(rows[b], acc.at[didx[b]], ssem[b], add=True)
        if do_meta:
            load_meta(i + DEPTH, b)

    for b in range(DEPTH):
        load_meta(b, b)
    for b in range(GA):
        wait_meta(b, b)
        start_gather(b)
    for i in range(DEPTH):
        step(i, i, wait_sc=(i >= DEPTH - GA))

    def _main(g, carry):
        i0 = DEPTH + g * DEPTH
        for b in range(DEPTH):
            step(i0 + b, b)
        return carry
    lax.fori_loop(0, (N_CHUNKS - 2 * DEPTH) // DEPTH, _main, 0)

    for k in range(DEPTH):
        i = N_CHUNKS - DEPTH + k
        step(i, i % DEPTH, next_gather=(k < DEPTH - GA), do_meta=False,
             sync_scatter=True)

    plsc.subcore_barrier()

    pltpu.sync_copy(acc.at[pl.ds(s * ROWS_PER_TILE, ROWS_PER_TILE)],
                    out.at[pl.ds(c * N_PAD + s * ROWS_PER_TILE, ROWS_PER_TILE)])


_layer = pl.kernel(
    _layer_body,
    out_type=jax.ShapeDtypeStruct((NC * N_PAD, HALF), jnp.float32),
    mesh=plsc.VectorSubcoreMesh(core_axis_name="c", subcore_axis_name="s",
                                num_cores=NC, num_subcores=NS),
    compiler_params=pltpu.CompilerParams(use_tc_tiling_on_sc=False,
                                         needs_layout_passes=False),
    scratch_types=(
        [pltpu.VMEM_SHARED((N_PAD, HALF), jnp.float32)]
        + [pltpu.VMEM((3, CHUNK), jnp.int32)] * DEPTH
        + [pltpu.VMEM((CHUNK, HALF), jnp.float32)] * DEPTH
        + [pltpu.VMEM((CHUNK,), jnp.int32)] * DEPTH
        + [pltpu.SemaphoreType.DMA] * (3 * DEPTH)
    ),
)


def _mean_body(t0, l1, l2, l3, out):
    out[...] = (t0[...] + l1[...] + l2[...] + l3[...]) * 0.25


_MBLK = 2176
_HOFF = N_PAD // _MBLK


def _mean(t0, l1, l2, l3, half):
    spec = pl.BlockSpec((_MBLK, HALF), lambda i, h=half: (i + h * _HOFF, 0))
    return pl.pallas_call(
        _mean_body,
        grid=(_HOFF,),
        in_specs=[spec] * 4,
        out_specs=pl.BlockSpec((_MBLK, HALF), lambda i: (i, 0)),
        out_shape=jax.ShapeDtypeStruct((N_PAD, HALF), jnp.float32),
    )(t0, l1, l2, l3)


def kernel(adj_indices, adj_values, user_table, item_table):
    dst = adj_indices[0].astype(jnp.int32)
    src = adj_indices[1].astype(jnp.int32)
    pad = E_PAD - N_EDGES
    src = jnp.pad(src, (0, pad))
    dst = jnp.pad(dst, (0, pad))
    val = jnp.pad(adj_values, (0, pad))
    valb = lax.bitcast_convert_type(val, jnp.int32)

    def mk(c):
        a = jnp.stack([src + c * N_PAD, dst, valb], axis=0)
        return a.reshape(3, CPT, CHUNK).transpose(1, 0, 2)
    packed = jnp.concatenate([mk(0), mk(1)], axis=0)

    zpad = jnp.zeros((N_PAD - N_NODES, HALF), jnp.float32)
    t0 = jnp.concatenate([user_table[:, :HALF], item_table[:, :HALF], zpad,
                          user_table[:, HALF:], item_table[:, HALF:], zpad],
                         axis=0)

    l1 = _layer(t0, packed)
    l2 = _layer(l1, packed)
    l3 = _layer(l2, packed)

    m0 = _mean(t0, l1, l2, l3, 0)
    m1 = _mean(t0, l1, l2, l3, 1)
    emb = jnp.concatenate([m0[:N_NODES], m1[:N_NODES]], axis=1)
    return emb[:N_USERS], emb[N_USERS:N_NODES]
